# baseline (speedup 1.0000x reference)
import functools

import jax
import jax.numpy as jnp
from jax import lax
from jax.experimental import pallas as pl
from jax.experimental.pallas import tpu as pltpu
from jax.experimental.pallas import tpu_sc as plsc

_D = 128
_L = 16
_C = 128
_NW = 32


_CB = 128


def _make_cast_kernel(v: int, r: int):
    nb_n = v // _CB
    nb_r = r // _CB
    nb = nb_n + nb_r
    bpw = -(-nb // _NW)
    bpw += bpw % 2
    tail_n = v - nb_n * _CB
    tail_r = r - nb_r * _CB
    mesh = plsc.VectorSubcoreMesh(core_axis_name="c", subcore_axis_name="s")

    @functools.partial(
        pl.kernel,
        out_type=[jax.ShapeDtypeStruct((v, _D), jnp.bfloat16),
                  jax.ShapeDtypeStruct((r, _D), jnp.bfloat16)],
        mesh=mesh,
        compiler_params=pltpu.CompilerParams(
            needs_layout_passes=False, use_tc_tiling_on_sc=False),
        scratch_types=[
            pltpu.VMEM((2, _CB, _D), jnp.float32),
            pltpu.VMEM((2, _CB, _D), jnp.bfloat16),
            pltpu.SemaphoreType.DMA,
            pltpu.SemaphoreType.DMA,
            pltpu.SemaphoreType.DMA,
            pltpu.SemaphoreType.DMA,
        ],
    )
    def cast_kernel(nodes_hbm, rel_hbm, n16_hbm, r16_hbm,
                    fin, fout, semi0, semi1, semo0, semo1):
        semi = (semi0, semi1)
        semo = (semo0, semo1)
        cid = lax.axis_index("c")
        sid = lax.axis_index("s")
        wid = sid * 2 + cid

        def block_of(j):
            return jnp.minimum(wid * bpw + j, nb - 1)

        def issue_in(j, b):
            g = block_of(j)
            is_nodes = g < nb_n

            @pl.when(is_nodes)
            def _():
                pltpu.async_copy(
                    nodes_hbm.at[pl.ds(g * _CB, _CB)], fin.at[b], semi[b])

            @pl.when(jnp.logical_not(is_nodes))
            def _():
                pltpu.async_copy(
                    rel_hbm.at[pl.ds((g - nb_n) * _CB, _CB)], fin.at[b], semi[b])

        def wait_in(b):
            pltpu.make_async_copy(
                nodes_hbm.at[pl.ds(0, _CB)], fin.at[b], semi[b]).wait()

        def convert(b, rows):
            def row_body(i, carry):
                for q in range(_D // (2 * _L)):
                    c0 = fin[b, i, pl.ds(q * 2 * _L, _L)]
                    c1 = fin[b, i, pl.ds(q * 2 * _L + _L, _L)]
                    fout[b, i, pl.ds(q * 2 * _L, 2 * _L)] = plsc.pack(
                        c0, c1, format=plsc.PackFormat.INTERLEAVED)
                return carry

            lax.fori_loop(0, rows, row_body, 0)

        def issue_out(j, b):
            g = block_of(j)
            is_nodes = g < nb_n

            @pl.when(is_nodes)
            def _():
                pltpu.async_copy(
                    fout.at[b], n16_hbm.at[pl.ds(g * _CB, _CB)], semo[b])

            @pl.when(jnp.logical_not(is_nodes))
            def _():
                pltpu.async_copy(
                    fout.at[b], r16_hbm.at[pl.ds((g - nb_n) * _CB, _CB)], semo[b])

        def wait_out(b):
            pltpu.make_async_copy(
                fout.at[b], n16_hbm.at[pl.ds(0, _CB)], semo[b]).wait()

        if tail_n:
            @pl.when(wid == _NW - 2)
            def _():
                pltpu.sync_copy(nodes_hbm.at[pl.ds(nb_n * _CB, tail_n)],
                                fin.at[0, pl.ds(0, tail_n)])
                convert(0, tail_n)
                pltpu.sync_copy(fout.at[0, pl.ds(0, tail_n)],
                                n16_hbm.at[pl.ds(nb_n * _CB, tail_n)])

        if tail_r:
            @pl.when(wid == _NW - 1)
            def _():
                pltpu.sync_copy(rel_hbm.at[pl.ds(nb_r * _CB, tail_r)],
                                fin.at[0, pl.ds(0, tail_r)])
                convert(0, tail_r)
                pltpu.sync_copy(fout.at[0, pl.ds(0, tail_r)],
                                r16_hbm.at[pl.ds(nb_r * _CB, tail_r)])

        issue_in(0, 0)
        issue_in(1, 1)

        def pair_body(cp, carry):
            j = cp * 2
            wait_in(0)

            @pl.when(cp > 0)
            def _():
                wait_out(0)

            convert(0, _CB)
            issue_in(j + 2, 0)
            issue_out(j, 0)
            wait_in(1)

            @pl.when(cp > 0)
            def _():
                wait_out(1)

            convert(1, _CB)
            issue_in(j + 3, 1)
            issue_out(j + 1, 1)
            return carry

        lax.fori_loop(0, bpw // 2, pair_body, 0)
        wait_in(0)
        wait_in(1)
        wait_out(0)
        wait_out(1)

    return cast_kernel


def _make_sc_kernel(n: int):
    n_chunks_total = -(-n // _C)
    cpw = -(-n_chunks_total // _NW)
    cpw = ((cpw + 3) // 4) * 4
    last_base = n - _C
    mesh = plsc.VectorSubcoreMesh(core_axis_name="c", subcore_axis_name="s")

    @functools.partial(
        pl.kernel,
        out_type=jax.ShapeDtypeStruct((n,), jnp.float32),
        mesh=mesh,
        compiler_params=pltpu.CompilerParams(
            needs_layout_passes=False, use_tc_tiling_on_sc=False),
        scratch_types=[
            pltpu.VMEM((4, _C), jnp.int32),
            pltpu.VMEM((4, _C), jnp.int32),
            pltpu.VMEM((4, _C), jnp.int32),
            pltpu.VMEM((4, _C, _D), jnp.bfloat16),
            pltpu.VMEM((4, _C, _D), jnp.bfloat16),
            pltpu.VMEM((4, _C, _D), jnp.bfloat16),
            pltpu.VMEM((_C,), jnp.float32),
            pltpu.SemaphoreType.DMA,
            pltpu.SemaphoreType.DMA,
            pltpu.SemaphoreType.DMA,
            pltpu.SemaphoreType.DMA,
            pltpu.SemaphoreType.DMA,
            pltpu.SemaphoreType.DMA,
            pltpu.SemaphoreType.DMA,
            pltpu.SemaphoreType.DMA,
        ],
    )
    def sc_kernel(sidx_hbm, pidx_hbm, oidx_hbm, nodes_hbm, rel_hbm, out_hbm,
                  sidx_v, pidx_v, oidx_v, s_v, p_v, o_v, out_v,
                  semt0, semt1, semt2, semt3, semr0, semr1, semr2, semr3):
        semt = (semt0, semt1, semt2, semt3)
        semr = (semr0, semr1, semr2, semr3)
        cid = lax.axis_index("c")
        sid = lax.axis_index("s")
        wid = sid * 2 + cid
        lanes = lax.iota(jnp.int32, _L)

        def chunk_base(j):
            return jnp.minimum((wid * cpw + j) * _C, last_base)

        def issue_trip(j, b):
            base = chunk_base(j)
            pltpu.async_copy(sidx_hbm.at[pl.ds(base, _C)], sidx_v.at[b], semt[b])
            pltpu.async_copy(pidx_hbm.at[pl.ds(base, _C)], pidx_v.at[b], semt[b])
            pltpu.async_copy(oidx_hbm.at[pl.ds(base, _C)], oidx_v.at[b], semt[b])

        def wait_trip(b):
            pltpu.make_async_copy(sidx_hbm.at[pl.ds(0, _C)], sidx_v.at[b], semt[b]).wait()
            pltpu.make_async_copy(pidx_hbm.at[pl.ds(0, _C)], pidx_v.at[b], semt[b]).wait()
            pltpu.make_async_copy(oidx_hbm.at[pl.ds(0, _C)], oidx_v.at[b], semt[b]).wait()

        def issue_rows(b):
            pltpu.async_copy(nodes_hbm.at[sidx_v.at[b]], s_v.at[b], semr[b])
            pltpu.async_copy(rel_hbm.at[pidx_v.at[b]], p_v.at[b], semr[b])
            pltpu.async_copy(nodes_hbm.at[oidx_v.at[b]], o_v.at[b], semr[b])

        def wait_rows(b):
            pltpu.make_async_copy(nodes_hbm.at[pl.ds(0, _C)], s_v.at[b], semr[b]).wait()
            pltpu.make_async_copy(rel_hbm.at[pl.ds(0, _C)], p_v.at[b], semr[b]).wait()
            pltpu.make_async_copy(nodes_hbm.at[pl.ds(0, _C)], o_v.at[b], semr[b]).wait()

        def compute(j, b):
            def group_body(g, carry2):
                gb = g * _L
                res = jnp.zeros((_L,), jnp.float32)
                for t in range(_L):
                    i = gb + t
                    acc = None
                    for dc in range(_D // (2 * _L)):
                        sl = pl.ds(dc * 2 * _L, 2 * _L)
                        s0, s1 = plsc.unpack(
                            s_v[b, i, sl], format=plsc.PackFormat.INTERLEAVED)
                        p0, p1 = plsc.unpack(
                            p_v[b, i, sl], format=plsc.PackFormat.INTERLEAVED)
                        o0, o1 = plsc.unpack(
                            o_v[b, i, sl], format=plsc.PackFormat.INTERLEAVED)
                        prod = s0 * p0 * o0 + s1 * p1 * o1
                        acc = prod if acc is None else acc + prod
                    res = jnp.where(lanes == t, jnp.sum(acc), res)
                out_v[pl.ds(gb, _L)] = res
                return carry2

            lax.fori_loop(0, _C // _L, group_body, 0)
            pltpu.sync_copy(out_v, out_hbm.at[pl.ds(chunk_base(j), _C)])

        for t in range(4):
            issue_trip(t, t)
        for t in range(3):
            wait_trip(t)
            issue_rows(t)

        def quad_body(q, carry):
            j0 = q * 4
            for t in range(4):
                j = j0 + t
                wait_trip((t + 3) % 4)
                issue_rows((t + 3) % 4)
                wait_rows(t)
                issue_trip(j + 4, t)
                compute(j, t)
            return carry

        lax.fori_loop(0, cpw // 4, quad_body, 0)
        wait_trip(3)
        wait_rows(0)
        wait_rows(1)
        wait_rows(2)

    return sc_kernel


def kernel(triples, nodes, relations):
    n = triples.shape[0]
    n16, r16 = _make_cast_kernel(nodes.shape[0], relations.shape[0])(
        nodes, relations)
    return _make_sc_kernel(n)(triples[:, 0], triples[:, 1], triples[:, 2],
                              n16, r16)


# speedup vs baseline: 1.0870x; 1.0870x over previous
import functools

import jax
import jax.numpy as jnp
from jax import lax
from jax.experimental import pallas as pl
from jax.experimental.pallas import tpu as pltpu
from jax.experimental.pallas import tpu_sc as plsc

_D = 128
_L = 16
_C = 128
_NW = 32


_CB = 128


def _make_cast_kernel(v: int, r: int):
    nb_n = v // _CB
    nb_r = r // _CB
    nb = nb_n + nb_r
    bpw = -(-nb // _NW)
    bpw += bpw % 2
    tail_n = v - nb_n * _CB
    tail_r = r - nb_r * _CB
    mesh = plsc.VectorSubcoreMesh(core_axis_name="c", subcore_axis_name="s")

    @functools.partial(
        pl.kernel,
        out_type=[jax.ShapeDtypeStruct((v, _D), jnp.bfloat16),
                  jax.ShapeDtypeStruct((r, _D), jnp.bfloat16)],
        mesh=mesh,
        compiler_params=pltpu.CompilerParams(
            needs_layout_passes=False, use_tc_tiling_on_sc=False),
        scratch_types=[
            pltpu.VMEM((2, _CB, _D), jnp.float32),
            pltpu.VMEM((2, _CB, _D), jnp.bfloat16),
            pltpu.SemaphoreType.DMA,
            pltpu.SemaphoreType.DMA,
            pltpu.SemaphoreType.DMA,
            pltpu.SemaphoreType.DMA,
        ],
    )
    def cast_kernel(nodes_hbm, rel_hbm, n16_hbm, r16_hbm,
                    fin, fout, semi0, semi1, semo0, semo1):
        semi = (semi0, semi1)
        semo = (semo0, semo1)
        cid = lax.axis_index("c")
        sid = lax.axis_index("s")
        wid = sid * 2 + cid

        def block_of(j):
            return jnp.minimum(wid * bpw + j, nb - 1)

        def issue_in(j, b):
            g = block_of(j)
            is_nodes = g < nb_n

            @pl.when(is_nodes)
            def _():
                pltpu.async_copy(
                    nodes_hbm.at[pl.ds(g * _CB, _CB)], fin.at[b], semi[b])

            @pl.when(jnp.logical_not(is_nodes))
            def _():
                pltpu.async_copy(
                    rel_hbm.at[pl.ds((g - nb_n) * _CB, _CB)], fin.at[b], semi[b])

        def wait_in(b):
            pltpu.make_async_copy(
                nodes_hbm.at[pl.ds(0, _CB)], fin.at[b], semi[b]).wait()

        def convert(b, rows, unroll=1):
            def row_body(i0, carry):
                for u in range(unroll):
                    i = i0 * unroll + u
                    for q in range(_D // (2 * _L)):
                        c0 = fin[b, i, pl.ds(q * 2 * _L, _L)]
                        c1 = fin[b, i, pl.ds(q * 2 * _L + _L, _L)]
                        fout[b, i, pl.ds(q * 2 * _L, 2 * _L)] = plsc.pack(
                            c0, c1, format=plsc.PackFormat.INTERLEAVED)
                return carry

            lax.fori_loop(0, rows // unroll, row_body, 0)

        def issue_out(j, b):
            g = block_of(j)
            is_nodes = g < nb_n

            @pl.when(is_nodes)
            def _():
                pltpu.async_copy(
                    fout.at[b], n16_hbm.at[pl.ds(g * _CB, _CB)], semo[b])

            @pl.when(jnp.logical_not(is_nodes))
            def _():
                pltpu.async_copy(
                    fout.at[b], r16_hbm.at[pl.ds((g - nb_n) * _CB, _CB)], semo[b])

        def wait_out(b):
            pltpu.make_async_copy(
                fout.at[b], n16_hbm.at[pl.ds(0, _CB)], semo[b]).wait()

        if tail_n:
            @pl.when(wid == _NW - 2)
            def _():
                pltpu.sync_copy(nodes_hbm.at[pl.ds(nb_n * _CB, tail_n)],
                                fin.at[0, pl.ds(0, tail_n)])
                convert(0, tail_n)
                pltpu.sync_copy(fout.at[0, pl.ds(0, tail_n)],
                                n16_hbm.at[pl.ds(nb_n * _CB, tail_n)])

        if tail_r:
            @pl.when(wid == _NW - 1)
            def _():
                pltpu.sync_copy(rel_hbm.at[pl.ds(nb_r * _CB, tail_r)],
                                fin.at[0, pl.ds(0, tail_r)])
                convert(0, tail_r)
                pltpu.sync_copy(fout.at[0, pl.ds(0, tail_r)],
                                r16_hbm.at[pl.ds(nb_r * _CB, tail_r)])

        issue_in(0, 0)
        issue_in(1, 1)

        def pair_body(cp, carry):
            j = cp * 2
            wait_in(0)

            @pl.when(cp > 0)
            def _():
                wait_out(0)

            convert(0, _CB, unroll=8)
            issue_in(j + 2, 0)
            issue_out(j, 0)
            wait_in(1)

            @pl.when(cp > 0)
            def _():
                wait_out(1)

            convert(1, _CB, unroll=8)
            issue_in(j + 3, 1)
            issue_out(j + 1, 1)
            return carry

        lax.fori_loop(0, bpw // 2, pair_body, 0)
        wait_in(0)
        wait_in(1)
        wait_out(0)
        wait_out(1)

    return cast_kernel


def _make_sc_kernel(n: int):
    n_chunks_total = -(-n // _C)
    cpw = -(-n_chunks_total // _NW)
    cpw += cpw % 2
    n_pairs = cpw // 2
    last_base = n - _C
    mesh = plsc.VectorSubcoreMesh(core_axis_name="c", subcore_axis_name="s")

    @functools.partial(
        pl.kernel,
        out_type=jax.ShapeDtypeStruct((n,), jnp.float32),
        mesh=mesh,
        compiler_params=pltpu.CompilerParams(
            needs_layout_passes=False, use_tc_tiling_on_sc=False),
        scratch_types=[
            pltpu.VMEM((2, _C), jnp.int32),
            pltpu.VMEM((2, _C), jnp.int32),
            pltpu.VMEM((2, _C), jnp.int32),
            pltpu.VMEM((2, _C, _D), jnp.bfloat16),
            pltpu.VMEM((2, _C, _D), jnp.bfloat16),
            pltpu.VMEM((2, _C, _D), jnp.bfloat16),
            pltpu.VMEM((_C,), jnp.float32),
            pltpu.SemaphoreType.DMA,
            pltpu.SemaphoreType.DMA,
            pltpu.SemaphoreType.DMA,
            pltpu.SemaphoreType.DMA,
        ],
    )
    def sc_kernel(sidx_hbm, pidx_hbm, oidx_hbm, nodes_hbm, rel_hbm, out_hbm,
                  sidx_v, pidx_v, oidx_v, s_v, p_v, o_v, out_v,
                  semt0, semt1, semr0, semr1):
        semt = (semt0, semt1)
        semr = (semr0, semr1)
        cid = lax.axis_index("c")
        sid = lax.axis_index("s")
        wid = sid * 2 + cid
        lanes = lax.iota(jnp.int32, _L)

        def chunk_base(j):
            return jnp.minimum((wid * cpw + j) * _C, last_base)

        def issue_trip(j, b):
            base = chunk_base(j)
            pltpu.async_copy(sidx_hbm.at[pl.ds(base, _C)], sidx_v.at[b], semt[b])
            pltpu.async_copy(pidx_hbm.at[pl.ds(base, _C)], pidx_v.at[b], semt[b])
            pltpu.async_copy(oidx_hbm.at[pl.ds(base, _C)], oidx_v.at[b], semt[b])

        def wait_trip(b):
            pltpu.make_async_copy(sidx_hbm.at[pl.ds(0, _C)], sidx_v.at[b], semt[b]).wait()
            pltpu.make_async_copy(pidx_hbm.at[pl.ds(0, _C)], pidx_v.at[b], semt[b]).wait()
            pltpu.make_async_copy(oidx_hbm.at[pl.ds(0, _C)], oidx_v.at[b], semt[b]).wait()

        def issue_rows(b):
            pltpu.async_copy(nodes_hbm.at[sidx_v.at[b]], s_v.at[b], semr[b])
            pltpu.async_copy(rel_hbm.at[pidx_v.at[b]], p_v.at[b], semr[b])
            pltpu.async_copy(nodes_hbm.at[oidx_v.at[b]], o_v.at[b], semr[b])

        def wait_rows(b):
            pltpu.make_async_copy(nodes_hbm.at[pl.ds(0, _C)], s_v.at[b], semr[b]).wait()
            pltpu.make_async_copy(rel_hbm.at[pl.ds(0, _C)], p_v.at[b], semr[b]).wait()
            pltpu.make_async_copy(nodes_hbm.at[pl.ds(0, _C)], o_v.at[b], semr[b]).wait()

        def compute(j, b):
            def group_body(g, carry2):
                gb = g * _L
                res = jnp.zeros((_L,), jnp.float32)
                for t in range(_L):
                    i = gb + t
                    acc = None
                    for dc in range(_D // (2 * _L)):
                        sl = pl.ds(dc * 2 * _L, 2 * _L)
                        s0, s1 = plsc.unpack(
                            s_v[b, i, sl], format=plsc.PackFormat.INTERLEAVED)
                        p0, p1 = plsc.unpack(
                            p_v[b, i, sl], format=plsc.PackFormat.INTERLEAVED)
                        o0, o1 = plsc.unpack(
                            o_v[b, i, sl], format=plsc.PackFormat.INTERLEAVED)
                        prod = s0 * p0 * o0 + s1 * p1 * o1
                        acc = prod if acc is None else acc + prod
                    res = jnp.where(lanes == t, jnp.sum(acc), res)
                out_v[pl.ds(gb, _L)] = res
                return carry2

            lax.fori_loop(0, _C // _L, group_body, 0)
            pltpu.sync_copy(out_v, out_hbm.at[pl.ds(chunk_base(j), _C)])

        issue_trip(0, 0)
        issue_trip(1, 1)
        wait_trip(0)
        issue_rows(0)

        def pair_body(cp, carry):
            j = cp * 2
            wait_trip(1)
            issue_rows(1)
            wait_rows(0)
            issue_trip(j + 2, 0)
            compute(j, 0)
            wait_trip(0)
            issue_rows(0)
            wait_rows(1)
            issue_trip(j + 3, 1)
            compute(j + 1, 1)
            return carry

        lax.fori_loop(0, n_pairs, pair_body, 0)
        wait_trip(1)
        wait_rows(0)

    return sc_kernel


def kernel(triples, nodes, relations):
    n = triples.shape[0]
    n16, r16 = _make_cast_kernel(nodes.shape[0], relations.shape[0])(
        nodes, relations)
    return _make_sc_kernel(n)(triples[:, 0], triples[:, 1], triples[:, 2],
                              n16, r16)
